# baseline (speedup 1.0000x reference)
import functools

import jax
import jax.numpy as jnp
from jax import lax
from jax.experimental import pallas as pl
from jax.experimental.pallas import tpu as pltpu
from jax.experimental.pallas import tpu_sc as plsc

_B, _T, _D, _M = 16, 512, 256, 4096
_HALF = _M // 2
_CH = 128
_NCH = _HALF // _CH
_ZROW = _B * _T
_MAXD = 8


def _lr_sc(table, dur):
    mesh = plsc.VectorSubcoreMesh(core_axis_name="c", subcore_axis_name="s")

    @functools.partial(
        pl.kernel,
        out_type=jax.ShapeDtypeStruct((_B * _M, _D), jnp.float32),
        mesh=mesh,
        compiler_params=pltpu.CompilerParams(needs_layout_passes=False),
        scratch_types=[
            pltpu.VMEM((_T,), jnp.int32),
            pltpu.VMEM((_HALF,), jnp.int32),
            pltpu.VMEM((2, _CH, _D), jnp.float32),
            pltpu.SemaphoreType.DMA,
            pltpu.SemaphoreType.DMA,
        ],
    )
    def lr(table_hbm, dur_hbm, out_hbm, dur_v, idx_v, rows_v, sem0, sem1):
        cid = lax.axis_index("c")
        sid = lax.axis_index("s")
        wid = sid * 2 + cid
        b = wid // 2
        m_base = (wid % 2) * _HALF

        pltpu.sync_copy(dur_hbm.at[b], dur_v)

        zrow = jnp.full((16,), _ZROW, jnp.int32)

        def _init(i, carry):
            idx_v[pl.ds(i * 16, 16)] = zrow
            return carry

        lax.fori_loop(0, _HALF // 16, _init, 0)

        lane = lax.broadcasted_iota(jnp.int32, (16,), 0)
        carry = jnp.int32(0)
        for j in range(_T // 16):
            d = dur_v[pl.ds(j * 16, 16)]
            end = plsc.cumsum(d) + carry
            start = end - d
            carry = jnp.max(end)
            jvec = b * _T + j * 16 + lane
            pos0 = start - m_base
            for k in range(_MAXD - 1):
                pos = pos0 + k
                valid = (d > k) & (pos >= 0) & (pos < _HALF)
                posc = jnp.clip(pos, 0, _HALF - 1)
                plsc.store_scatter(idx_v, [posc], jvec, mask=valid)

        sems = (sem0, sem1)

        def _gather(ci, buf):
            return pltpu.async_copy(
                table_hbm.at[idx_v.at[pl.ds(ci * _CH, _CH)]],
                rows_v.at[buf],
                sems[buf],
            )

        cps = [None, None]
        cps[0] = _gather(0, 0)
        row0 = b * _M + m_base
        for ci in range(_NCH):
            buf = ci % 2
            if ci + 1 < _NCH:
                cps[1 - buf] = _gather(ci + 1, 1 - buf)
            cps[buf].wait()
            pltpu.sync_copy(
                rows_v.at[buf], out_hbm.at[pl.ds(row0 + ci * _CH, _CH)]
            )

    return lr(table, dur)


def _shift_down(a):
    return jnp.concatenate(
        [jnp.zeros((1, a.shape[1]), a.dtype), a[:-1, :]], axis=0
    )


def _shift_up(a):
    return jnp.concatenate(
        [a[1:, :], jnp.zeros((1, a.shape[1]), a.dtype)], axis=0
    )


def _dp_body(x_ref, w1_ref, b1_ref, g1_ref, be1_ref, w2_ref, b2_ref, g2_ref,
             be2_ref, wl_ref, bl_ref, o_ref):
    x = x_ref[0]

    def conv_ln(h, w_ref, bias, gain, beta):
        y = jnp.dot(h, w_ref[1], preferred_element_type=jnp.float32)
        y = y + _shift_down(
            jnp.dot(h, w_ref[0], preferred_element_type=jnp.float32)
        )
        y = y + _shift_up(
            jnp.dot(h, w_ref[2], preferred_element_type=jnp.float32)
        )
        y = jnp.maximum(y + bias, 0.0)
        m = jnp.mean(y, axis=-1, keepdims=True)
        yc = y - m
        v = jnp.mean(yc * yc, axis=-1, keepdims=True)
        return yc * jax.lax.rsqrt(v + 1e-5) * gain + beta

    h = conv_ln(x, w1_ref, b1_ref[...], g1_ref[...], be1_ref[...])
    h = conv_ln(h, w2_ref, b2_ref[...], g2_ref[...], be2_ref[...])
    dp = jnp.sum(h * wl_ref[...], axis=-1, keepdims=True) + bl_ref[...]
    o_ref[...] = jnp.maximum(dp, 0.0).reshape(1, _T, 1)


def _dp_tc(x, w1t, b1, g1, be1, w2t, b2, g2, be2, wl, bl):
    vec = pl.BlockSpec((1, _D), lambda i: (0, 0))
    return pl.pallas_call(
        _dp_body,
        grid=(_B,),
        in_specs=[
            pl.BlockSpec((1, _T, _D), lambda i: (i, 0, 0)),
            pl.BlockSpec((3, _D, _D), lambda i: (0, 0, 0)),
            vec, vec, vec,
            pl.BlockSpec((3, _D, _D), lambda i: (0, 0, 0)),
            vec, vec, vec,
            vec,
            pl.BlockSpec((1, 1), lambda i: (0, 0)),
        ],
        out_specs=pl.BlockSpec((1, _T, 1), lambda i: (i, 0, 0)),
        out_shape=jax.ShapeDtypeStruct((_B, _T, 1), jnp.float32),
    )(x, w1t, b1, g1, be1, w2t, b2, g2, be2, wl, bl)


def kernel(x, W1, b1, g1, be1, W2, b2, g2, be2, Wl, bl, target, mel_max_length):
    w1t = jnp.transpose(W1, (2, 1, 0))
    w2t = jnp.transpose(W2, (2, 1, 0))
    dp = _dp_tc(
        x, w1t, b1.reshape(1, _D), g1.reshape(1, _D), be1.reshape(1, _D),
        w2t, b2.reshape(1, _D), g2.reshape(1, _D), be2.reshape(1, _D),
        Wl.reshape(1, _D), bl.reshape(1, 1),
    )[..., 0]
    table = jnp.concatenate(
        [x.reshape(_B * _T, _D), jnp.zeros((8, _D), x.dtype)], axis=0
    )
    out = _lr_sc(table, target.astype(jnp.int32)).reshape(_B, _M, _D)
    return (out, dp)


# speedup vs baseline: 7.7557x; 7.7557x over previous
import functools

import jax
import jax.numpy as jnp
from jax import lax
from jax.experimental import pallas as pl
from jax.experimental.pallas import tpu as pltpu
from jax.experimental.pallas import tpu_sc as plsc

_B, _T, _D, _M = 16, 512, 256, 4096
_HALF = _M // 2
_CH = 128
_NCH = _HALF // _CH
_NZPAD = 128
_ZROW = _B * _T
_MAXD = 8


def _lr_sc(table, dur):
    mesh = plsc.VectorSubcoreMesh(core_axis_name="c", subcore_axis_name="s")

    @functools.partial(
        pl.kernel,
        out_type=jax.ShapeDtypeStruct((_B * _M, _D), jnp.float32),
        mesh=mesh,
        compiler_params=pltpu.CompilerParams(
            needs_layout_passes=False, use_tc_tiling_on_sc=False
        ),
        scratch_types=[
            pltpu.VMEM((_T,), jnp.int32),
            pltpu.VMEM((_HALF,), jnp.int32),
            pltpu.VMEM((2, _CH, _D), jnp.float32),
            pltpu.SemaphoreType.DMA,
            pltpu.SemaphoreType.DMA,
        ],
    )
    def lr(table_hbm, dur_hbm, out_hbm, dur_v, idx_v, rows_v, sem0, sem1):
        cid = lax.axis_index("c")
        sid = lax.axis_index("s")
        wid = cid * 16 + sid
        b = wid // 2
        m_base = (wid % 2) * _HALF

        pltpu.sync_copy(dur_hbm.at[b], dur_v)

        lane = lax.broadcasted_iota(jnp.int32, (16,), 0)

        def _init(i, carry):
            idx_v[pl.ds(i * 16, 16)] = (
                _ZROW + (wid * 4 + i * 16 + lane) % _NZPAD
            )
            return carry

        lax.fori_loop(0, _HALF // 16, _init, 0)

        carry = jnp.int32(0)
        for j in range(_T // 16):
            d = dur_v[pl.ds(j * 16, 16)]
            end = plsc.cumsum(d) + carry
            start = end - d
            carry = jnp.max(end)
            jvec = b * _T + j * 16 + lane
            pos0 = start - m_base
            for k in range(_MAXD - 1):
                pos = pos0 + k
                valid = (d > k) & (pos >= 0) & (pos < _HALF)
                posc = jnp.clip(pos, 0, _HALF - 1)
                plsc.store_scatter(idx_v, [posc], jvec, mask=valid)

        sems = (sem0, sem1)

        def _gather(ci, buf):
            return pltpu.async_copy(
                table_hbm.at[idx_v.at[pl.ds(ci * _CH, _CH)]],
                rows_v.at[buf],
                sems[buf],
            )

        cps = [None, None]
        cps[0] = _gather(0, 0)
        row0 = b * _M + m_base
        for ci in range(_NCH):
            buf = ci % 2
            if ci + 1 < _NCH:
                cps[1 - buf] = _gather(ci + 1, 1 - buf)
            cps[buf].wait()
            pltpu.sync_copy(
                rows_v.at[buf], out_hbm.at[pl.ds(row0 + ci * _CH, _CH)]
            )

    return lr(table, dur)


def _shift_down(a):
    return jnp.concatenate(
        [jnp.zeros((1, a.shape[1]), a.dtype), a[:-1, :]], axis=0
    )


def _shift_up(a):
    return jnp.concatenate(
        [a[1:, :], jnp.zeros((1, a.shape[1]), a.dtype)], axis=0
    )


def _dp_body(x_ref, w1_ref, b1_ref, g1_ref, be1_ref, w2_ref, b2_ref, g2_ref,
             be2_ref, wl_ref, bl_ref, o_ref):
    x = x_ref[0]

    def conv_ln(h, w_ref, bias, gain, beta):
        y = jnp.dot(h, w_ref[1], preferred_element_type=jnp.float32)
        y = y + _shift_down(
            jnp.dot(h, w_ref[0], preferred_element_type=jnp.float32)
        )
        y = y + _shift_up(
            jnp.dot(h, w_ref[2], preferred_element_type=jnp.float32)
        )
        y = jnp.maximum(y + bias, 0.0)
        m = jnp.mean(y, axis=-1, keepdims=True)
        yc = y - m
        v = jnp.mean(yc * yc, axis=-1, keepdims=True)
        return yc * jax.lax.rsqrt(v + 1e-5) * gain + beta

    h = conv_ln(x, w1_ref, b1_ref[...], g1_ref[...], be1_ref[...])
    h = conv_ln(h, w2_ref, b2_ref[...], g2_ref[...], be2_ref[...])
    dp = jnp.sum(h * wl_ref[...], axis=-1, keepdims=True) + bl_ref[...]
    o_ref[...] = jnp.maximum(dp, 0.0).reshape(1, _T, 1)


def _dp_tc(x, w1t, b1, g1, be1, w2t, b2, g2, be2, wl, bl):
    vec = pl.BlockSpec((1, _D), lambda i: (0, 0))
    return pl.pallas_call(
        _dp_body,
        grid=(_B,),
        in_specs=[
            pl.BlockSpec((1, _T, _D), lambda i: (i, 0, 0)),
            pl.BlockSpec((3, _D, _D), lambda i: (0, 0, 0)),
            vec, vec, vec,
            pl.BlockSpec((3, _D, _D), lambda i: (0, 0, 0)),
            vec, vec, vec,
            vec,
            pl.BlockSpec((1, 1), lambda i: (0, 0)),
        ],
        out_specs=pl.BlockSpec((1, _T, 1), lambda i: (i, 0, 0)),
        out_shape=jax.ShapeDtypeStruct((_B, _T, 1), jnp.float32),
    )(x, w1t, b1, g1, be1, w2t, b2, g2, be2, wl, bl)


def kernel(x, W1, b1, g1, be1, W2, b2, g2, be2, Wl, bl, target, mel_max_length):
    w1t = jnp.transpose(W1, (2, 1, 0))
    w2t = jnp.transpose(W2, (2, 1, 0))
    dp = _dp_tc(
        x, w1t, b1.reshape(1, _D), g1.reshape(1, _D), be1.reshape(1, _D),
        w2t, b2.reshape(1, _D), g2.reshape(1, _D), be2.reshape(1, _D),
        Wl.reshape(1, _D), bl.reshape(1, 1),
    )[..., 0]
    table = jnp.concatenate(
        [x.reshape(_B * _T, _D), jnp.zeros((_NZPAD, _D), x.dtype)], axis=0
    )
    out = _lr_sc(table, target.astype(jnp.int32)).reshape(_B, _M, _D)
    return (out, dp)


# speedup vs baseline: 18.6913x; 2.4100x over previous
import functools

import jax
import jax.numpy as jnp
from jax import lax
from jax.experimental import pallas as pl
from jax.experimental.pallas import tpu as pltpu
from jax.experimental.pallas import tpu_sc as plsc

_B, _T, _D, _M = 16, 512, 256, 4096
_HALF = _M // 2
_CH = 128
_NCH = _HALF // _CH
_NZPAD = 128
_ZROW = _B * _T
_MAXD = 8


def _lr_sc(table_l, table_r, dur):
    mesh = plsc.VectorSubcoreMesh(core_axis_name="c", subcore_axis_name="s")

    @functools.partial(
        pl.kernel,
        out_type=jax.ShapeDtypeStruct((_B, _M, _D), jnp.float32),
        mesh=mesh,
        compiler_params=pltpu.CompilerParams(needs_layout_passes=False),
        scratch_types=[
            pltpu.VMEM((_T,), jnp.int32),
            pltpu.VMEM((_HALF,), jnp.int32),
            pltpu.VMEM((2, _CH, 128), jnp.float32),
            pltpu.VMEM((2, _CH, 128), jnp.float32),
            pltpu.VMEM((_CH, 128), jnp.float32),
            pltpu.SemaphoreType.DMA,
            pltpu.SemaphoreType.DMA,
        ],
    )
    def lr(tl_hbm, tr_hbm, dur_hbm, out_hbm, dur_v, idx_v, bufl, bufr, zbuf,
           sem0, sem1):
        cid = lax.axis_index("c")
        sid = lax.axis_index("s")
        wid = cid * 16 + sid
        b = wid // 2
        m_base = (wid % 2) * _HALF

        pltpu.sync_copy(dur_hbm.at[b], dur_v)

        lane = lax.broadcasted_iota(jnp.int32, (16,), 0)
        zeros16 = jnp.zeros((16,), jnp.float32)

        def _initz(i, carry):
            zbuf[i // 8, pl.ds((i % 8) * 16, 16)] = zeros16
            return carry

        lax.fori_loop(0, _CH * 8, _initz, 0)

        def _init(i, carry):
            idx_v[pl.ds(i * 16, 16)] = (
                _ZROW + (wid * 4 + i * 16 + lane) % _NZPAD
            )
            return carry

        lax.fori_loop(0, _HALF // 16, _init, 0)

        carry = jnp.int32(0)
        for j in range(_T // 16):
            d = dur_v[pl.ds(j * 16, 16)]
            end = plsc.cumsum(d) + carry
            start = end - d
            carry = jnp.max(end)
            jvec = b * _T + j * 16 + lane
            pos0 = start - m_base
            for k in range(_MAXD - 1):
                pos = pos0 + k
                valid = (d > k) & (pos >= 0) & (pos < _HALF)
                posc = jnp.clip(pos, 0, _HALF - 1)
                plsc.store_scatter(idx_v, [posc], jvec, mask=valid)

        nreal = carry - m_base
        sems = (sem0, sem1)

        def _cps(ci, ph):
            idxs = idx_v.at[pl.ds(ci * _CH, _CH)]
            return (
                pltpu.make_async_copy(tl_hbm.at[idxs], bufl.at[ph], sems[ph]),
                pltpu.make_async_copy(tr_hbm.at[idxs], bufr.at[ph], sems[ph]),
            )

        def _issue(ci, ph):
            @pl.when(nreal > ci * _CH)
            def _():
                cl, cr = _cps(ci, ph)
                cl.start()
                cr.start()

        _issue(0, 0)
        for ci in range(_NCH):
            ph = ci % 2
            if ci + 1 < _NCH:
                _issue(ci + 1, 1 - ph)
            r0 = m_base + ci * _CH
            real = nreal > ci * _CH

            @pl.when(real)
            def _():
                cl, cr = _cps(ci, ph)
                cl.wait()
                cr.wait()
                pltpu.sync_copy(
                    bufl.at[ph], out_hbm.at[b, pl.ds(r0, _CH), pl.ds(0, 128)]
                )
                pltpu.sync_copy(
                    bufr.at[ph], out_hbm.at[b, pl.ds(r0, _CH), pl.ds(128, 128)]
                )

            @pl.when(jnp.logical_not(real))
            def _():
                pltpu.sync_copy(
                    zbuf, out_hbm.at[b, pl.ds(r0, _CH), pl.ds(0, 128)]
                )
                pltpu.sync_copy(
                    zbuf, out_hbm.at[b, pl.ds(r0, _CH), pl.ds(128, 128)]
                )

    return lr(table_l, table_r, dur)


def _shift_down(a):
    return jnp.concatenate(
        [jnp.zeros((1, a.shape[1]), a.dtype), a[:-1, :]], axis=0
    )


def _shift_up(a):
    return jnp.concatenate(
        [a[1:, :], jnp.zeros((1, a.shape[1]), a.dtype)], axis=0
    )


def _dp_body(x_ref, w1_ref, b1_ref, g1_ref, be1_ref, w2_ref, b2_ref, g2_ref,
             be2_ref, wl_ref, bl_ref, o_ref):
    x = x_ref[0]

    def conv_ln(h, w_ref, bias, gain, beta):
        y = jnp.dot(h, w_ref[1], preferred_element_type=jnp.float32)
        y = y + _shift_down(
            jnp.dot(h, w_ref[0], preferred_element_type=jnp.float32)
        )
        y = y + _shift_up(
            jnp.dot(h, w_ref[2], preferred_element_type=jnp.float32)
        )
        y = jnp.maximum(y + bias, 0.0)
        m = jnp.mean(y, axis=-1, keepdims=True)
        yc = y - m
        v = jnp.mean(yc * yc, axis=-1, keepdims=True)
        return yc * jax.lax.rsqrt(v + 1e-5) * gain + beta

    h = conv_ln(x, w1_ref, b1_ref[...], g1_ref[...], be1_ref[...])
    h = conv_ln(h, w2_ref, b2_ref[...], g2_ref[...], be2_ref[...])
    dp = jnp.sum(h * wl_ref[...], axis=-1, keepdims=True) + bl_ref[...]
    o_ref[...] = jnp.maximum(dp, 0.0).reshape(1, _T, 1)


def _dp_tc(x, w1t, b1, g1, be1, w2t, b2, g2, be2, wl, bl):
    vec = pl.BlockSpec((1, _D), lambda i: (0, 0))
    return pl.pallas_call(
        _dp_body,
        grid=(_B,),
        in_specs=[
            pl.BlockSpec((1, _T, _D), lambda i: (i, 0, 0)),
            pl.BlockSpec((3, _D, _D), lambda i: (0, 0, 0)),
            vec, vec, vec,
            pl.BlockSpec((3, _D, _D), lambda i: (0, 0, 0)),
            vec, vec, vec,
            vec,
            pl.BlockSpec((1, 1), lambda i: (0, 0)),
        ],
        out_specs=pl.BlockSpec((1, _T, 1), lambda i: (i, 0, 0)),
        out_shape=jax.ShapeDtypeStruct((_B, _T, 1), jnp.float32),
    )(x, w1t, b1, g1, be1, w2t, b2, g2, be2, wl, bl)


def kernel(x, W1, b1, g1, be1, W2, b2, g2, be2, Wl, bl, target, mel_max_length):
    w1t = jnp.transpose(W1, (2, 1, 0))
    w2t = jnp.transpose(W2, (2, 1, 0))
    dp = _dp_tc(
        x, w1t, b1.reshape(1, _D), g1.reshape(1, _D), be1.reshape(1, _D),
        w2t, b2.reshape(1, _D), g2.reshape(1, _D), be2.reshape(1, _D),
        Wl.reshape(1, _D), bl.reshape(1, 1),
    )[..., 0]
    xf = x.reshape(_B * _T, _D)
    zpad = jnp.zeros((_NZPAD, 128), x.dtype)
    table_l = jnp.concatenate([xf[:, :128], zpad], axis=0)
    table_r = jnp.concatenate([xf[:, 128:], zpad], axis=0)
    out = _lr_sc(table_l, table_r, target.astype(jnp.int32))
    return (out, dp)


# speedup vs baseline: 18.7693x; 1.0042x over previous
import functools

import jax
import jax.numpy as jnp
from jax import lax
from jax.experimental import pallas as pl
from jax.experimental.pallas import tpu as pltpu
from jax.experimental.pallas import tpu_sc as plsc

_B, _T, _D, _M = 16, 512, 256, 4096
_HALF = _M // 2
_CH = 128
_NCH = _HALF // _CH
_NZPAD = 128
_ZROW = _B * _T
_MAXD = 8


def _lr_sc(table_l, table_r, dur):
    mesh = plsc.VectorSubcoreMesh(core_axis_name="c", subcore_axis_name="s")

    @functools.partial(
        pl.kernel,
        out_type=jax.ShapeDtypeStruct((_B, _M, _D), jnp.float32),
        mesh=mesh,
        compiler_params=pltpu.CompilerParams(needs_layout_passes=False),
        scratch_types=[
            pltpu.VMEM((_T,), jnp.int32),
            pltpu.VMEM((_HALF,), jnp.int32),
            pltpu.VMEM((2, _CH, 128), jnp.float32),
            pltpu.VMEM((2, _CH, 128), jnp.float32),
            pltpu.VMEM((_CH, 128), jnp.float32),
            pltpu.SemaphoreType.DMA,
            pltpu.SemaphoreType.DMA,
            pltpu.SemaphoreType.DMA,
            pltpu.SemaphoreType.DMA,
            pltpu.SemaphoreType.DMA,
        ],
    )
    def lr(tl_hbm, tr_hbm, dur_hbm, out_hbm, dur_v, idx_v, bufl, bufr, zbuf,
           semg0, semg1, semwl, semwr, semz):
        cid = lax.axis_index("c")
        sid = lax.axis_index("s")
        wid = cid * 16 + sid
        b = wid // 2
        m_base = (wid % 2) * _HALF

        pltpu.sync_copy(dur_hbm.at[b], dur_v)

        lane = lax.broadcasted_iota(jnp.int32, (16,), 0)
        zeros16 = jnp.zeros((16,), jnp.float32)

        def _initz(i, carry):
            zbuf[i // 8, pl.ds((i % 8) * 16, 16)] = zeros16
            return carry

        lax.fori_loop(0, _CH * 8, _initz, 0)

        def _init(i, carry):
            idx_v[pl.ds(i * 16, 16)] = (
                _ZROW + (wid * 4 + i * 16 + lane) % _NZPAD
            )
            return carry

        lax.fori_loop(0, _HALF // 16, _init, 0)

        carry = jnp.int32(0)
        for j in range(_T // 16):
            d = dur_v[pl.ds(j * 16, 16)]
            end = plsc.cumsum(d) + carry
            start = end - d
            carry = jnp.max(end)
            jvec = b * _T + j * 16 + lane
            pos0 = start - m_base
            for k in range(_MAXD - 1):
                pos = pos0 + k
                valid = (d > k) & (pos >= 0) & (pos < _HALF)
                posc = jnp.clip(pos, 0, _HALF - 1)
                plsc.store_scatter(idx_v, [posc], jvec, mask=valid)

        nreal = carry - m_base
        semg = (semg0, semg1)

        def _real(ci):
            return nreal > ci * _CH

        def _gather_cps(ci, ph):
            idxs = idx_v.at[pl.ds(ci * _CH, _CH)]
            return (
                pltpu.make_async_copy(tl_hbm.at[idxs], bufl.at[ph], semg[ph]),
                pltpu.make_async_copy(tr_hbm.at[idxs], bufr.at[ph], semg[ph]),
            )

        def _write_cps(ci, ph):
            r0 = m_base + ci * _CH
            return (
                pltpu.make_async_copy(
                    bufl.at[ph],
                    out_hbm.at[b, pl.ds(r0, _CH), pl.ds(0, 128)],
                    semwl,
                ),
                pltpu.make_async_copy(
                    bufr.at[ph],
                    out_hbm.at[b, pl.ds(r0, _CH), pl.ds(128, 128)],
                    semwr,
                ),
            )

        def _zero_cps(ci):
            r0 = m_base + ci * _CH
            return (
                pltpu.make_async_copy(
                    zbuf, out_hbm.at[b, pl.ds(r0, _CH), pl.ds(0, 128)], semz
                ),
                pltpu.make_async_copy(
                    zbuf, out_hbm.at[b, pl.ds(r0, _CH), pl.ds(128, 128)], semz
                ),
            )

        def _issue_gather(ci, ph):
            @pl.when(_real(ci))
            def _():
                cl, cr = _gather_cps(ci, ph)
                cl.start()
                cr.start()

        _issue_gather(0, 0)
        for ci in range(_NCH):
            ph = ci % 2
            if ci + 1 < _NCH:
                if ci >= 1:
                    @pl.when(_real(ci - 1))
                    def _():
                        wl, wr = _write_cps(ci - 1, 1 - ph)
                        wl.wait()
                        wr.wait()

                _issue_gather(ci + 1, 1 - ph)

            @pl.when(_real(ci))
            def _():
                cl, cr = _gather_cps(ci, ph)
                cl.wait()
                cr.wait()
                wl, wr = _write_cps(ci, ph)
                wl.start()
                wr.start()

            @pl.when(jnp.logical_not(_real(ci)))
            def _():
                zl, zr = _zero_cps(ci)
                zl.start()
                zr.start()

        for ci in (_NCH - 2, _NCH - 1):

            @pl.when(_real(ci))
            def _():
                wl, wr = _write_cps(ci, ci % 2)
                wl.wait()
                wr.wait()

        for ci in range(_NCH):

            @pl.when(jnp.logical_not(_real(ci)))
            def _():
                zl, zr = _zero_cps(ci)
                zl.wait()
                zr.wait()

    return lr(table_l, table_r, dur)


def _shift_down(a):
    return jnp.concatenate(
        [jnp.zeros((1, a.shape[1]), a.dtype), a[:-1, :]], axis=0
    )


def _shift_up(a):
    return jnp.concatenate(
        [a[1:, :], jnp.zeros((1, a.shape[1]), a.dtype)], axis=0
    )


def _dp_body(x_ref, w1_ref, b1_ref, g1_ref, be1_ref, w2_ref, b2_ref, g2_ref,
             be2_ref, wl_ref, bl_ref, o_ref):
    x = x_ref[0]

    def conv_ln(h, w_ref, bias, gain, beta):
        y = jnp.dot(h, w_ref[1], preferred_element_type=jnp.float32)
        y = y + _shift_down(
            jnp.dot(h, w_ref[0], preferred_element_type=jnp.float32)
        )
        y = y + _shift_up(
            jnp.dot(h, w_ref[2], preferred_element_type=jnp.float32)
        )
        y = jnp.maximum(y + bias, 0.0)
        m = jnp.mean(y, axis=-1, keepdims=True)
        yc = y - m
        v = jnp.mean(yc * yc, axis=-1, keepdims=True)
        return yc * jax.lax.rsqrt(v + 1e-5) * gain + beta

    h = conv_ln(x, w1_ref, b1_ref[...], g1_ref[...], be1_ref[...])
    h = conv_ln(h, w2_ref, b2_ref[...], g2_ref[...], be2_ref[...])
    dp = jnp.sum(h * wl_ref[...], axis=-1, keepdims=True) + bl_ref[...]
    o_ref[...] = jnp.maximum(dp, 0.0).reshape(1, _T, 1)


def _dp_tc(x, w1t, b1, g1, be1, w2t, b2, g2, be2, wl, bl):
    vec = pl.BlockSpec((1, _D), lambda i: (0, 0))
    return pl.pallas_call(
        _dp_body,
        grid=(_B,),
        in_specs=[
            pl.BlockSpec((1, _T, _D), lambda i: (i, 0, 0)),
            pl.BlockSpec((3, _D, _D), lambda i: (0, 0, 0)),
            vec, vec, vec,
            pl.BlockSpec((3, _D, _D), lambda i: (0, 0, 0)),
            vec, vec, vec,
            vec,
            pl.BlockSpec((1, 1), lambda i: (0, 0)),
        ],
        out_specs=pl.BlockSpec((1, _T, 1), lambda i: (i, 0, 0)),
        out_shape=jax.ShapeDtypeStruct((_B, _T, 1), jnp.float32),
    )(x, w1t, b1, g1, be1, w2t, b2, g2, be2, wl, bl)


def kernel(x, W1, b1, g1, be1, W2, b2, g2, be2, Wl, bl, target, mel_max_length):
    w1t = jnp.transpose(W1, (2, 1, 0))
    w2t = jnp.transpose(W2, (2, 1, 0))
    dp = _dp_tc(
        x, w1t, b1.reshape(1, _D), g1.reshape(1, _D), be1.reshape(1, _D),
        w2t, b2.reshape(1, _D), g2.reshape(1, _D), be2.reshape(1, _D),
        Wl.reshape(1, _D), bl.reshape(1, 1),
    )[..., 0]
    xf = x.reshape(_B * _T, _D)
    zpad = jnp.zeros((_NZPAD, 128), x.dtype)
    table_l = jnp.concatenate([xf[:, :128], zpad], axis=0)
    table_r = jnp.concatenate([xf[:, 128:], zpad], axis=0)
    out = _lr_sc(table_l, table_r, target.astype(jnp.int32))
    return (out, dp)


# speedup vs baseline: 20.1978x; 1.0761x over previous
import functools

import jax
import jax.numpy as jnp
from jax import lax
from jax.experimental import pallas as pl
from jax.experimental.pallas import tpu as pltpu
from jax.experimental.pallas import tpu_sc as plsc

_B, _T, _D, _M = 16, 512, 256, 4096
_HALF = _M // 2
_CH = 128
_NCH = _HALF // _CH
_NCHA = 24
_NZPAD = 128
_ZROW = _B * _T
_MAXD = 8


def _lr_sc(table_l, table_r, dur):
    mesh = plsc.VectorSubcoreMesh(core_axis_name="c", subcore_axis_name="s")

    @functools.partial(
        pl.kernel,
        out_type=jax.ShapeDtypeStruct((_B, _M, _D), jnp.float32),
        mesh=mesh,
        compiler_params=pltpu.CompilerParams(needs_layout_passes=False),
        scratch_types=[
            pltpu.VMEM((_T,), jnp.int32),
            pltpu.VMEM((_HALF,), jnp.int32),
            pltpu.VMEM((2, _CH, 128), jnp.float32),
            pltpu.VMEM((2, _CH, 128), jnp.float32),
            pltpu.VMEM((_CH, 128), jnp.float32),
            pltpu.SemaphoreType.DMA,
            pltpu.SemaphoreType.DMA,
            pltpu.SemaphoreType.DMA,
            pltpu.SemaphoreType.DMA,
            pltpu.SemaphoreType.DMA,
        ],
    )
    def lr(tl_hbm, tr_hbm, dur_hbm, out_hbm, dur_v, idx_v, bufl, bufr, zbuf,
           semg0, semg1, semwl, semwr, semz):
        cid = lax.axis_index("c")
        sid = lax.axis_index("s")
        wid = cid * 16 + sid
        b = wid // 2
        half = wid % 2

        pltpu.sync_copy(dur_hbm.at[b], dur_v)

        lane = lax.broadcasted_iota(jnp.int32, (16,), 0)
        zeros16 = jnp.zeros((16,), jnp.float32)

        def _initz(i, carry):
            zbuf[i // 8, pl.ds((i % 8) * 16, 16)] = zeros16
            return carry

        lax.fori_loop(0, _CH * 8, _initz, 0)

        def _init(i, carry):
            idx_v[pl.ds(i * 16, 16)] = (
                _ZROW + (wid * 4 + i * 16 + lane) % _NZPAD
            )
            return carry

        lax.fori_loop(0, _HALF // 16, _init, 0)

        carry = jnp.int32(0)
        for j in range(_T // 16):
            carry = carry + jnp.max(plsc.cumsum(dur_v[pl.ds(j * 16, 16)]))
        total = carry

        split = (total // 512) * 128 + 1024
        m_base = half * split
        m_end = jnp.where(half == 0, split, _M)

        carry = jnp.int32(0)
        for j in range(_T // 16):
            d = dur_v[pl.ds(j * 16, 16)]
            end = plsc.cumsum(d) + carry
            start = end - d
            carry = jnp.max(end)
            jvec = b * _T + j * 16 + lane
            pos0 = start - m_base
            for k in range(_MAXD - 1):
                pos = pos0 + k
                valid = (d > k) & (pos >= 0) & (pos < _HALF)
                posc = jnp.clip(pos, 0, _HALF - 1)
                plsc.store_scatter(idx_v, [posc], jvec, mask=valid)

        nreal = jnp.minimum(total, m_end) - m_base
        nact = m_end - m_base
        semg = (semg0, semg1)

        def _real(ci):
            return nreal > ci * _CH

        def _zerow(ci):
            return jnp.logical_and(nact > ci * _CH, nreal <= ci * _CH)

        def _gather_cps(ci, ph):
            idxs = idx_v.at[pl.ds(ci * _CH, _CH)]
            return (
                pltpu.make_async_copy(tl_hbm.at[idxs], bufl.at[ph], semg[ph]),
                pltpu.make_async_copy(tr_hbm.at[idxs], bufr.at[ph], semg[ph]),
            )

        def _write_cps(ci, ph):
            r0 = m_base + ci * _CH
            return (
                pltpu.make_async_copy(
                    bufl.at[ph],
                    out_hbm.at[b, pl.ds(r0, _CH), pl.ds(0, 128)],
                    semwl,
                ),
                pltpu.make_async_copy(
                    bufr.at[ph],
                    out_hbm.at[b, pl.ds(r0, _CH), pl.ds(128, 128)],
                    semwr,
                ),
            )

        def _zero_cps(ci):
            r0 = m_base + ci * _CH
            return (
                pltpu.make_async_copy(
                    zbuf, out_hbm.at[b, pl.ds(r0, _CH), pl.ds(0, 128)], semz
                ),
                pltpu.make_async_copy(
                    zbuf, out_hbm.at[b, pl.ds(r0, _CH), pl.ds(128, 128)], semz
                ),
            )

        def _issue_gather(ci, ph):
            @pl.when(_real(ci))
            def _():
                cl, cr = _gather_cps(ci, ph)
                cl.start()
                cr.start()

        _issue_gather(0, 0)
        for ci in range(_NCHA):
            ph = ci % 2
            if ci + 1 < _NCH:
                if ci >= 1:
                    @pl.when(_real(ci - 1))
                    def _():
                        wl, wr = _write_cps(ci - 1, 1 - ph)
                        wl.wait()
                        wr.wait()

                _issue_gather(ci + 1, 1 - ph)

            if ci < _NCH:

                @pl.when(_real(ci))
                def _():
                    cl, cr = _gather_cps(ci, ph)
                    cl.wait()
                    cr.wait()
                    wl, wr = _write_cps(ci, ph)
                    wl.start()
                    wr.start()

            @pl.when(_zerow(ci))
            def _():
                zl, zr = _zero_cps(ci)
                zl.start()
                zr.start()

        for ci in (_NCH - 2, _NCH - 1):

            @pl.when(_real(ci))
            def _():
                wl, wr = _write_cps(ci, ci % 2)
                wl.wait()
                wr.wait()

        for ci in range(_NCHA):

            @pl.when(_zerow(ci))
            def _():
                zl, zr = _zero_cps(ci)
                zl.wait()
                zr.wait()

    return lr(table_l, table_r, dur)


def _shift_down(a):
    return jnp.concatenate(
        [jnp.zeros((1, a.shape[1]), a.dtype), a[:-1, :]], axis=0
    )


def _shift_up(a):
    return jnp.concatenate(
        [a[1:, :], jnp.zeros((1, a.shape[1]), a.dtype)], axis=0
    )


def _dp_body(x_ref, w1_ref, b1_ref, g1_ref, be1_ref, w2_ref, b2_ref, g2_ref,
             be2_ref, wl_ref, bl_ref, o_ref):
    x = x_ref[0]

    def conv_ln(h, w_ref, bias, gain, beta):
        y = jnp.dot(h, w_ref[1], preferred_element_type=jnp.float32)
        y = y + _shift_down(
            jnp.dot(h, w_ref[0], preferred_element_type=jnp.float32)
        )
        y = y + _shift_up(
            jnp.dot(h, w_ref[2], preferred_element_type=jnp.float32)
        )
        y = jnp.maximum(y + bias, 0.0)
        m = jnp.mean(y, axis=-1, keepdims=True)
        yc = y - m
        v = jnp.mean(yc * yc, axis=-1, keepdims=True)
        return yc * jax.lax.rsqrt(v + 1e-5) * gain + beta

    h = conv_ln(x, w1_ref, b1_ref[...], g1_ref[...], be1_ref[...])
    h = conv_ln(h, w2_ref, b2_ref[...], g2_ref[...], be2_ref[...])
    dp = jnp.sum(h * wl_ref[...], axis=-1) + bl_ref[0, 0]
    o_ref[...] = jnp.maximum(dp, 0.0).reshape(1, 1, _T)


def _dp_tc(x, w1t, b1, g1, be1, w2t, b2, g2, be2, wl, bl):
    vec = pl.BlockSpec((1, _D), lambda i: (0, 0))
    return pl.pallas_call(
        _dp_body,
        grid=(_B,),
        in_specs=[
            pl.BlockSpec((1, _T, _D), lambda i: (i, 0, 0)),
            pl.BlockSpec((3, _D, _D), lambda i: (0, 0, 0)),
            vec, vec, vec,
            pl.BlockSpec((3, _D, _D), lambda i: (0, 0, 0)),
            vec, vec, vec,
            vec,
            pl.BlockSpec((1, 1), lambda i: (0, 0)),
        ],
        out_specs=pl.BlockSpec((1, 1, _T), lambda i: (i, 0, 0)),
        out_shape=jax.ShapeDtypeStruct((_B, 1, _T), jnp.float32),
    )(x, w1t, b1, g1, be1, w2t, b2, g2, be2, wl, bl)


def kernel(x, W1, b1, g1, be1, W2, b2, g2, be2, Wl, bl, target, mel_max_length):
    w1t = jnp.transpose(W1, (2, 1, 0))
    w2t = jnp.transpose(W2, (2, 1, 0))
    dp = _dp_tc(
        x, w1t, b1.reshape(1, _D), g1.reshape(1, _D), be1.reshape(1, _D),
        w2t, b2.reshape(1, _D), g2.reshape(1, _D), be2.reshape(1, _D),
        Wl.reshape(1, _D), bl.reshape(1, 1),
    )[:, 0, :]
    xf = x.reshape(_B * _T, _D)
    zpad = jnp.zeros((_NZPAD, 128), x.dtype)
    table_l = jnp.concatenate([xf[:, :128], zpad], axis=0)
    table_r = jnp.concatenate([xf[:, 128:], zpad], axis=0)
    out = _lr_sc(table_l, table_r, target.astype(jnp.int32))
    return (out, dp)


# speedup vs baseline: 20.4822x; 1.0141x over previous
import functools

import jax
import jax.numpy as jnp
from jax import lax
from jax.experimental import pallas as pl
from jax.experimental.pallas import tpu as pltpu
from jax.experimental.pallas import tpu_sc as plsc

_B, _T, _D, _M = 16, 512, 256, 4096
_HALF = _M // 2
_CH = 128
_NCH = _HALF // _CH
_NCHA = 24
_NZPAD = 128
_ZROW = _B * _T
_MAXD = 8


def _lr_sc(table, dur):
    mesh = plsc.VectorSubcoreMesh(core_axis_name="c", subcore_axis_name="s")

    @functools.partial(
        pl.kernel,
        out_type=jax.ShapeDtypeStruct((_B, _M, _D), jnp.float32),
        mesh=mesh,
        compiler_params=pltpu.CompilerParams(needs_layout_passes=False),
        scratch_types=[
            pltpu.VMEM((_T,), jnp.int32),
            pltpu.VMEM((_HALF,), jnp.int32),
            pltpu.VMEM((2, _CH, 128), jnp.float32),
            pltpu.VMEM((2, _CH, 128), jnp.float32),
            pltpu.VMEM((_CH, 128), jnp.float32),
            pltpu.SemaphoreType.DMA,
            pltpu.SemaphoreType.DMA,
            pltpu.SemaphoreType.DMA,
            pltpu.SemaphoreType.DMA,
            pltpu.SemaphoreType.DMA,
        ],
    )
    def lr(tab_hbm, dur_hbm, out_hbm, dur_v, idx_v, bufl, bufr, zbuf,
           semg0, semg1, semwl, semwr, semz):
        cid = lax.axis_index("c")
        sid = lax.axis_index("s")
        wid = cid * 16 + sid
        b = wid // 2
        half = wid % 2

        pltpu.sync_copy(dur_hbm.at[b], dur_v)

        lane = lax.broadcasted_iota(jnp.int32, (16,), 0)
        zeros16 = jnp.zeros((16,), jnp.float32)

        def _initz(i, carry):
            zbuf[i // 8, pl.ds((i % 8) * 16, 16)] = zeros16
            return carry

        lax.fori_loop(0, _CH * 8, _initz, 0)

        def _init(i, carry):
            idx_v[pl.ds(i * 16, 16)] = (
                _ZROW + (wid * 4 + i * 16 + lane) % _NZPAD
            )
            return carry

        lax.fori_loop(0, _HALF // 16, _init, 0)

        carry = jnp.int32(0)
        for j in range(_T // 16):
            carry = carry + jnp.max(plsc.cumsum(dur_v[pl.ds(j * 16, 16)]))
        total = carry

        split = (total // 512) * 128 + 1024
        m_base = half * split
        m_end = jnp.where(half == 0, split, _M)

        carry = jnp.int32(0)
        for j in range(_T // 16):
            d = dur_v[pl.ds(j * 16, 16)]
            end = plsc.cumsum(d) + carry
            start = end - d
            carry = jnp.max(end)
            jvec = b * _T + j * 16 + lane
            pos0 = start - m_base
            for k in range(_MAXD - 1):
                pos = pos0 + k
                valid = (d > k) & (pos >= 0) & (pos < _HALF)
                posc = jnp.clip(pos, 0, _HALF - 1)
                plsc.store_scatter(idx_v, [posc], jvec, mask=valid)

        nreal = jnp.minimum(total, m_end) - m_base
        nact = m_end - m_base
        semg = (semg0, semg1)

        def _real(ci):
            return nreal > ci * _CH

        def _zerow(ci):
            return jnp.logical_and(nact > ci * _CH, nreal <= ci * _CH)

        def _gather_cps(ci, ph):
            idxs = idx_v.at[pl.ds(ci * _CH, _CH)]
            return (
                pltpu.make_async_copy(
                    tab_hbm.at[0].at[idxs], bufl.at[ph], semg[ph]
                ),
                pltpu.make_async_copy(
                    tab_hbm.at[1].at[idxs], bufr.at[ph], semg[ph]
                ),
            )

        def _write_cps(ci, ph):
            r0 = m_base + ci * _CH
            return (
                pltpu.make_async_copy(
                    bufl.at[ph],
                    out_hbm.at[b, pl.ds(r0, _CH), pl.ds(0, 128)],
                    semwl,
                ),
                pltpu.make_async_copy(
                    bufr.at[ph],
                    out_hbm.at[b, pl.ds(r0, _CH), pl.ds(128, 128)],
                    semwr,
                ),
            )

        def _zero_cps(ci):
            r0 = m_base + ci * _CH
            return (
                pltpu.make_async_copy(
                    zbuf, out_hbm.at[b, pl.ds(r0, _CH), pl.ds(0, 128)], semz
                ),
                pltpu.make_async_copy(
                    zbuf, out_hbm.at[b, pl.ds(r0, _CH), pl.ds(128, 128)], semz
                ),
            )

        def _issue_gather(ci, ph):
            @pl.when(_real(ci))
            def _():
                cl, cr = _gather_cps(ci, ph)
                cl.start()
                cr.start()

        _issue_gather(0, 0)
        for ci in range(_NCHA):
            ph = ci % 2
            if ci + 1 < _NCH:
                if ci >= 1:
                    @pl.when(_real(ci - 1))
                    def _():
                        wl, wr = _write_cps(ci - 1, 1 - ph)
                        wl.wait()
                        wr.wait()

                _issue_gather(ci + 1, 1 - ph)

            if ci < _NCH:

                @pl.when(_real(ci))
                def _():
                    cl, cr = _gather_cps(ci, ph)
                    cl.wait()
                    cr.wait()
                    wl, wr = _write_cps(ci, ph)
                    wl.start()
                    wr.start()

            @pl.when(_zerow(ci))
            def _():
                zl, zr = _zero_cps(ci)
                zl.start()
                zr.start()

        for ci in (_NCH - 2, _NCH - 1):

            @pl.when(_real(ci))
            def _():
                wl, wr = _write_cps(ci, ci % 2)
                wl.wait()
                wr.wait()

        for ci in range(_NCHA):

            @pl.when(_zerow(ci))
            def _():
                zl, zr = _zero_cps(ci)
                zl.wait()
                zr.wait()

    return lr(table, dur)


_BB = 4


def _shift_down(a, mask):
    s = jnp.concatenate(
        [jnp.zeros((1, a.shape[1]), a.dtype), a[:-1, :]], axis=0
    )
    return jnp.where(mask, s, 0.0)


def _shift_up(a, mask):
    s = jnp.concatenate(
        [a[1:, :], jnp.zeros((1, a.shape[1]), a.dtype)], axis=0
    )
    return jnp.where(mask, s, 0.0)


def _dp_body(x_ref, w1_ref, b1_ref, g1_ref, be1_ref, w2_ref, b2_ref, g2_ref,
             be2_ref, wl_ref, bl_ref, o_ref):
    n = _BB * _T
    x = x_ref[...].reshape(n, _D)
    row = lax.broadcasted_iota(jnp.int32, (n, 1), 0)
    mdown = (row % _T) != 0
    mup = (row % _T) != (_T - 1)

    def conv_ln(h, w_ref, bias, gain, beta):
        y = jnp.dot(h, w_ref[1], preferred_element_type=jnp.float32)
        y = y + _shift_down(
            jnp.dot(h, w_ref[0], preferred_element_type=jnp.float32), mdown
        )
        y = y + _shift_up(
            jnp.dot(h, w_ref[2], preferred_element_type=jnp.float32), mup
        )
        y = jnp.maximum(y + bias, 0.0)
        m = jnp.mean(y, axis=-1, keepdims=True)
        yc = y - m
        v = jnp.mean(yc * yc, axis=-1, keepdims=True)
        return yc * jax.lax.rsqrt(v + 1e-5) * gain + beta

    h = conv_ln(x, w1_ref, b1_ref[...], g1_ref[...], be1_ref[...])
    h = conv_ln(h, w2_ref, b2_ref[...], g2_ref[...], be2_ref[...])
    dp = jnp.sum(h * wl_ref[...], axis=-1) + bl_ref[0, 0]
    o_ref[...] = jnp.maximum(dp, 0.0).reshape(_BB, 1, _T)


def _dp_tc(x, w1t, b1, g1, be1, w2t, b2, g2, be2, wl, bl):
    vec = pl.BlockSpec((1, _D), lambda i: (0, 0))
    return pl.pallas_call(
        _dp_body,
        grid=(_B // _BB,),
        in_specs=[
            pl.BlockSpec((_BB, _T, _D), lambda i: (i, 0, 0)),
            pl.BlockSpec((3, _D, _D), lambda i: (0, 0, 0)),
            vec, vec, vec,
            pl.BlockSpec((3, _D, _D), lambda i: (0, 0, 0)),
            vec, vec, vec,
            vec,
            pl.BlockSpec((1, 1), lambda i: (0, 0)),
        ],
        out_specs=pl.BlockSpec((_BB, 1, _T), lambda i: (i, 0, 0)),
        out_shape=jax.ShapeDtypeStruct((_B, 1, _T), jnp.float32),
    )(x, w1t, b1, g1, be1, w2t, b2, g2, be2, wl, bl)


def kernel(x, W1, b1, g1, be1, W2, b2, g2, be2, Wl, bl, target, mel_max_length):
    w1t = jnp.transpose(W1, (2, 1, 0))
    w2t = jnp.transpose(W2, (2, 1, 0))
    dp = _dp_tc(
        x, w1t, b1.reshape(1, _D), g1.reshape(1, _D), be1.reshape(1, _D),
        w2t, b2.reshape(1, _D), g2.reshape(1, _D), be2.reshape(1, _D),
        Wl.reshape(1, _D), bl.reshape(1, 1),
    )[:, 0, :]
    xf = x.reshape(_B * _T, _D)
    table = jnp.pad(
        jnp.stack([xf[:, :128], xf[:, 128:]], axis=0),
        ((0, 0), (0, _NZPAD), (0, 0)),
    )
    out = _lr_sc(table, target.astype(jnp.int32))
    return (out, dp)
